# baseline (speedup 1.0000x reference)
import jax
import jax.numpy as jnp
from jax.experimental import pallas as pl
from jax.experimental.pallas import tpu as pltpu

_T = 56


def _body(x_ref, o_ref):
    x = x_ref[...].reshape(_T, 2, 64, 224)
    xa = x[:, :, :, 0:128]
    xb = x[:, :, :, 128:224]

    def _idx(n, off):
        base = jax.lax.broadcasted_iota(jnp.int32, (_T, 2, 64, n), 3)
        return 2 * base + off

    for wt, (src, n) in enumerate(((xa, 64), (xb, 48))):
        for kw in (0, 1):
            g = jnp.take_along_axis(src, _idx(n, kw), axis=3)
            for kh in (0, 1):
                q0 = (kh * 2 + kw) * 64
                o_ref[0, :, q0 : q0 + 64, wt * 64 : wt * 64 + n] = g[:, kh]


def kernel(batch):
    B, H, W, C = batch.shape
    k = 2
    Ho, Wo = H // k, W // k
    Co = k * k * C

    vi = jnp.transpose(batch, (0, 1, 3, 2))

    nchunk = 4
    Bc = B // nchunk
    parts = []
    for kk in range(nchunk):
        vo = pl.pallas_call(
            _body,
            grid=(Bc, Ho // _T),
            in_specs=[
                pl.BlockSpec(
                    (1, k * _T, C, W),
                    lambda b, i, kk=kk: (b + Bc * kk, i, 0, 0),
                ),
            ],
            out_specs=pl.BlockSpec((1, _T, Co, Wo), lambda b, i: (b, i, 0, 0)),
            out_shape=jax.ShapeDtypeStruct((Bc, Ho, Co, Wo), batch.dtype),
            compiler_params=pltpu.CompilerParams(
                dimension_semantics=("parallel", "arbitrary"),
            ),
        )(vi)
        parts.append(jnp.transpose(vo, (0, 1, 3, 2)))
    return jnp.concatenate(parts, axis=0)


# speedup vs baseline: 1.4624x; 1.4624x over previous
import jax
import jax.numpy as jnp
from jax.experimental import pallas as pl
from jax.experimental.pallas import tpu as pltpu

_T = 56


def _body(x_ref, o_ref):
    x = x_ref[...].reshape(_T, 2, 64, 224)
    xa = x[:, :, :, 0:128]
    xb = x[:, :, :, 128:224]

    def _idx(n, off):
        base = jax.lax.broadcasted_iota(jnp.int32, (_T, 2, 64, n), 3)
        return 2 * base + off

    for wt, (src, n) in enumerate(((xa, 64), (xb, 48))):
        for kw in (0, 1):
            g = jnp.take_along_axis(src, _idx(n, kw), axis=3)
            for kh in (0, 1):
                q0 = (kh * 2 + kw) * 64
                o_ref[0, :, q0 : q0 + 64, wt * 64 : wt * 64 + n] = g[:, kh]


def kernel(batch):
    B, H, W, C = batch.shape
    k = 2
    Ho, Wo = H // k, W // k
    Co = k * k * C

    vi = jnp.transpose(batch, (0, 1, 3, 2))

    vo = pl.pallas_call(
        _body,
        grid=(B, Ho // _T),
        in_specs=[
            pl.BlockSpec((1, k * _T, C, W), lambda b, i: (b, i, 0, 0)),
        ],
        out_specs=pl.BlockSpec((1, _T, Co, Wo), lambda b, i: (b, i, 0, 0)),
        out_shape=jax.ShapeDtypeStruct((B, Ho, Co, Wo), batch.dtype),
        compiler_params=pltpu.CompilerParams(
            dimension_semantics=("parallel", "arbitrary"),
        ),
    )(vi)
    return jnp.transpose(vo, (0, 1, 3, 2))
